# baseline (speedup 1.0000x reference)
import functools
import math

import jax
import jax.numpy as jnp
from jax import lax
from jax.experimental import pallas as pl
from jax.experimental.pallas import tpu as pltpu
from jax.experimental.pallas import tpu_sc as plsc

_N = 10000
_E = 320000
_D = 128
_EPS = 1e-5

_NC = 2
_NS = 16
_NW = _NC * _NS
_C = 66
_CHT = 312
_NCHUNKS = _NS * _CHT
_EPAD = _NCHUNKS * _C
_NR = 4
_NI = 6
_UNROLL = 12
_NACC = 10240
_ZROWS = _NACC // _NS


def _sc_segment_sum(feat, edge_r, d_ext):
    ncol16 = d_ext // 16
    mesh = plsc.VectorSubcoreMesh(core_axis_name="c", subcore_axis_name="s",
                                  num_cores=_NC, num_subcores=_NS)

    def body(feat_hbm, edge_hbm, out_hbm,
             i0, i1, i2, i3, i4, i5,
             r0, r1, r2, r3, acc_sh,
             si0, si1, si2, si3, si4, si5,
             sr0, sr1, sr2, sr3):
        cid = lax.axis_index("c")
        sid = lax.axis_index("s")

        @pl.when(cid == 0)
        def _sc0():
            cbase = sid * _CHT
            idx = [i0, i1, i2, i3, i4, i5]
            isem = [si0, si1, si2, si3, si4, si5]
            rows = [r0, r1, r2, r3]
            rsem = [sr0, sr1, sr2, sr3]

            def zrow(r, carry):
                for c in range(ncol16):
                    r0[r, pl.ds(c * 16, 16)] = jnp.zeros((16,),
                                                         jnp.float32)
                return carry
            lax.fori_loop(0, _C, zrow, 0)
            base = sid * _ZROWS
            for j in range(_ZROWS // _C):
                pltpu.sync_copy(r0, acc_sh.at[pl.ds(base + j * _C, _C)])
            _rem = _ZROWS % _C
            if _rem:
                pltpu.sync_copy(
                    r0.at[pl.ds(0, _rem)],
                    acc_sh.at[pl.ds(base + (_ZROWS // _C) * _C, _rem)])
            plsc.subcore_barrier()

            def load_idx(j, slot):
                pltpu.async_copy(edge_hbm.at[cbase + j], idx[slot],
                                 isem[slot])

            def wait_idx(j, slot):
                pltpu.make_async_copy(edge_hbm.at[cbase + j], idx[slot],
                                      isem[slot]).wait()

            def gather(slot_i, slot_r):
                pltpu.async_copy(feat_hbm.at[idx[slot_i].at[0]],
                                 rows[slot_r], rsem[slot_r])

            def wait_gather(slot_i, slot_r):
                pltpu.make_async_copy(feat_hbm.at[idx[slot_i].at[0]],
                                      rows[slot_r], rsem[slot_r]).wait()

            for j in range(_NI - 1):
                load_idx(j, j)
            for j in range(_NR - 1):
                wait_idx(j, j)
                gather(j, j)

            def step(it, carry):
                for k in range(_UNROLL):
                    jv = it * _UNROLL + k
                    s_i = k % _NI
                    s_g = (k + _NR - 1) % _NI
                    s_l = (k + _NI - 1) % _NI
                    s_r = k % _NR

                    wait_gather(s_i, s_r)

                    @pl.when(jv + _NR - 1 < _CHT)
                    def _():
                        wait_idx(jv + _NR - 1, s_g)
                        gather(s_g, (k + _NR - 1) % _NR)

                    @pl.when(jv + _NI - 1 < _CHT)
                    def _():
                        load_idx(jv + _NI - 1, s_l)

                    pltpu.sync_copy(rows[s_r], acc_sh.at[idx[s_i].at[1]],
                                    add=True)
                return carry
            lax.fori_loop(0, _CHT // _UNROLL, step, 0)
            plsc.subcore_barrier()

            pltpu.sync_copy(acc_sh.at[pl.ds(base, _ZROWS)],
                            out_hbm.at[pl.ds(base, _ZROWS)])

    return pl.kernel(
        body,
        out_type=jax.ShapeDtypeStruct((_NACC, d_ext), jnp.float32),
        mesh=mesh,
        scratch_types=[
            pltpu.VMEM((2, _C), jnp.int32),
            pltpu.VMEM((2, _C), jnp.int32),
            pltpu.VMEM((2, _C), jnp.int32),
            pltpu.VMEM((2, _C), jnp.int32),
            pltpu.VMEM((2, _C), jnp.int32),
            pltpu.VMEM((2, _C), jnp.int32),
            pltpu.VMEM((_C, d_ext), jnp.float32),
            pltpu.VMEM((_C, d_ext), jnp.float32),
            pltpu.VMEM((_C, d_ext), jnp.float32),
            pltpu.VMEM((_C, d_ext), jnp.float32),
            pltpu.VMEM_SHARED((_NACC, d_ext), jnp.float32),
            pltpu.SemaphoreType.DMA,
            pltpu.SemaphoreType.DMA,
            pltpu.SemaphoreType.DMA,
            pltpu.SemaphoreType.DMA,
            pltpu.SemaphoreType.DMA,
            pltpu.SemaphoreType.DMA,
            pltpu.SemaphoreType.DMA,
            pltpu.SemaphoreType.DMA,
            pltpu.SemaphoreType.DMA,
            pltpu.SemaphoreType.DMA,
        ],
        compiler_params=pltpu.CompilerParams(use_tc_tiling_on_sc=False),
    )(feat, edge_r)


_BLK = 512
_INV_STD = 1.0 / math.sqrt(1.0 + _EPS)


def _l1_body(p_ref, x_ref, wn_ref, wr_ref, b_ref, g_ref, be_ref,
             h_ref, rdeg_ref):
    acc = p_ref[...]
    rdeg = 1.0 / jnp.maximum(acc[:, _D:_D + 1], 1.0)
    mean = acc[:, :_D] * rdeg
    z = (jnp.dot(mean, wn_ref[...], preferred_element_type=jnp.float32)
         + jnp.dot(x_ref[...], wr_ref[...], preferred_element_type=jnp.float32)
         + b_ref[...])
    z = z * (_INV_STD * g_ref[...]) + be_ref[...]
    h_ref[...] = jnp.maximum(z, 0.0)
    rdeg_ref[...] = jnp.broadcast_to(rdeg, (_BLK, _D))


def _tc_layer1(p, x_pad, w1nT, w1rT, b1, gamma1, beta1):
    return pl.pallas_call(
        _l1_body,
        grid=(_NACC // _BLK,),
        in_specs=[
            pl.BlockSpec((_BLK, _D + 16), lambda i: (i, 0)),
            pl.BlockSpec((_BLK, _D), lambda i: (i, 0)),
            pl.BlockSpec((_D, _D), lambda i: (0, 0)),
            pl.BlockSpec((_D, _D), lambda i: (0, 0)),
            pl.BlockSpec((1, _D), lambda i: (0, 0)),
            pl.BlockSpec((1, _D), lambda i: (0, 0)),
            pl.BlockSpec((1, _D), lambda i: (0, 0)),
        ],
        out_specs=[
            pl.BlockSpec((_BLK, _D), lambda i: (i, 0)),
            pl.BlockSpec((_BLK, _D), lambda i: (i, 0)),
        ],
        out_shape=[
            jax.ShapeDtypeStruct((_NACC, _D), jnp.float32),
            jax.ShapeDtypeStruct((_NACC, _D), jnp.float32),
        ],
    )(p, x_pad, w1nT, w1rT, b1, gamma1, beta1)


def _l2_body(p_ref, h_ref, rdeg_ref, wn_ref, wr_ref, b_ref, out_ref):
    mean = p_ref[...] * rdeg_ref[...]
    z = (jnp.dot(mean, wn_ref[...], preferred_element_type=jnp.float32)
         + jnp.dot(h_ref[...], wr_ref[...], preferred_element_type=jnp.float32)
         + b_ref[...])
    m = jnp.max(z, axis=1, keepdims=True)
    s = jnp.sum(jnp.exp(z - m), axis=1, keepdims=True)
    out_ref[...] = z - m - jnp.log(s)


def _tc_layer2(p2, h, rdeg, w2nT, w2rT, b2):
    return pl.pallas_call(
        _l2_body,
        grid=(_NACC // _BLK,),
        in_specs=[
            pl.BlockSpec((_BLK, _D), lambda i: (i, 0)),
            pl.BlockSpec((_BLK, _D), lambda i: (i, 0)),
            pl.BlockSpec((_BLK, _D), lambda i: (i, 0)),
            pl.BlockSpec((_D, _D), lambda i: (0, 0)),
            pl.BlockSpec((_D, _D), lambda i: (0, 0)),
            pl.BlockSpec((1, _D), lambda i: (0, 0)),
        ],
        out_specs=pl.BlockSpec((_BLK, _D), lambda i: (i, 0)),
        out_shape=jax.ShapeDtypeStruct((_NACC, _D), jnp.float32),
    )(p2, h, rdeg, w2nT, w2rT, b2)


def kernel(x, edge_index, W1n, W1r, b1, gamma1, beta1, W2n, W2r, b2):
    src = edge_index[0]
    dst = edge_index[1]
    pad = _EPAD - _E
    src_r = jnp.concatenate(
        [src, jnp.zeros((pad,), jnp.int32)]).reshape(_NCHUNKS, _C)
    dst_r = jnp.concatenate(
        [dst, jnp.full((pad,), _N, jnp.int32)]).reshape(_NCHUNKS, _C)
    edge_r = jnp.stack([src_r, dst_r], axis=1)
    x_ext = jnp.concatenate(
        [x, jnp.ones((_N, 1), jnp.float32), jnp.zeros((_N, 15), jnp.float32)],
        axis=1)

    p1 = _sc_segment_sum(x_ext, edge_r, _D + 16)
    x_pad = jnp.pad(x, ((0, _NACC - _N), (0, 0)))
    h, rdeg = _tc_layer1(p1, x_pad, W1n.T, W1r.T, b1.reshape(1, _D),
                         gamma1.reshape(1, _D), beta1.reshape(1, _D))
    p2 = _sc_segment_sum(h, edge_r, _D)
    out = _tc_layer2(p2, h, rdeg, W2n.T, W2r.T, b2.reshape(1, _D))
    return out[:_N]


# speedup vs baseline: 2.4985x; 2.4985x over previous
import functools
import math

import jax
import jax.numpy as jnp
from jax import lax
from jax.experimental import pallas as pl
from jax.experimental.pallas import tpu as pltpu
from jax.experimental.pallas import tpu_sc as plsc

_N = 10000
_E = 320000
_D = 128
_EPS = 1e-5

_NC = 2
_NS = 16
_NW = _NC * _NS
_C = 88
_CHT = 228
_NCHUNKS = _NS * _CHT
_EPAD = _NCHUNKS * _C
_NR = 3
_NI = 4
_UNROLL = 12
_NACC = 10240
_ZROWS = _NACC // _NS


def _sc_segment_sum(feat, edge_r, d_ext):
    ncol16 = d_ext // 16
    mesh = plsc.VectorSubcoreMesh(core_axis_name="c", subcore_axis_name="s",
                                  num_cores=_NC, num_subcores=_NS)

    def body(feat_hbm, edge_hbm, out_hbm,
             i0, i1, i2, i3,
             r0, r1, r2, acc_sh,
             si0, si1, si2, si3,
             sr0, sr1, sr2):
        cid = lax.axis_index("c")
        sid = lax.axis_index("s")

        @pl.when(cid == 0)
        def _sc0():
            cbase = sid * _CHT
            idx = [i0, i1, i2, i3]
            isem = [si0, si1, si2, si3]
            rows = [r0, r1, r2]
            rsem = [sr0, sr1, sr2]

            def zrow(r, carry):
                for c in range(ncol16):
                    r0[r, pl.ds(c * 16, 16)] = jnp.zeros((16,),
                                                         jnp.float32)
                return carry
            lax.fori_loop(0, _C, zrow, 0)
            base = sid * _ZROWS
            for j in range(_ZROWS // _C):
                pltpu.sync_copy(r0, acc_sh.at[pl.ds(base + j * _C, _C)])
            _rem = _ZROWS % _C
            if _rem:
                pltpu.sync_copy(
                    r0.at[pl.ds(0, _rem)],
                    acc_sh.at[pl.ds(base + (_ZROWS // _C) * _C, _rem)])
            plsc.subcore_barrier()

            def load_idx(j, slot):
                pltpu.async_copy(edge_hbm.at[cbase + j], idx[slot],
                                 isem[slot])

            def wait_idx(j, slot):
                pltpu.make_async_copy(edge_hbm.at[cbase + j], idx[slot],
                                      isem[slot]).wait()

            def gather(slot_i, slot_r):
                pltpu.async_copy(feat_hbm.at[idx[slot_i].at[0]],
                                 rows[slot_r], rsem[slot_r])

            def wait_gather(slot_i, slot_r):
                pltpu.make_async_copy(feat_hbm.at[idx[slot_i].at[0]],
                                      rows[slot_r], rsem[slot_r]).wait()

            for j in range(_NI - 1):
                load_idx(j, j)
            for j in range(_NR - 1):
                wait_idx(j, j)
                gather(j, j)

            def step(it, carry):
                for k in range(_UNROLL):
                    jv = it * _UNROLL + k
                    s_i = k % _NI
                    s_g = (k + _NR - 1) % _NI
                    s_l = (k + _NI - 1) % _NI
                    s_r = k % _NR

                    wait_gather(s_i, s_r)

                    @pl.when(jv + _NR - 1 < _CHT)
                    def _():
                        wait_idx(jv + _NR - 1, s_g)
                        gather(s_g, (k + _NR - 1) % _NR)

                    @pl.when(jv + _NI - 1 < _CHT)
                    def _():
                        load_idx(jv + _NI - 1, s_l)

                    pltpu.sync_copy(rows[s_r], acc_sh.at[idx[s_i].at[1]],
                                    add=True)
                return carry
            lax.fori_loop(0, _CHT // _UNROLL, step, 0)
            plsc.subcore_barrier()

            pltpu.sync_copy(acc_sh.at[pl.ds(base, _ZROWS)],
                            out_hbm.at[pl.ds(base, _ZROWS)])

    return pl.kernel(
        body,
        out_type=jax.ShapeDtypeStruct((_NACC, d_ext), jnp.float32),
        mesh=mesh,
        scratch_types=[
            pltpu.VMEM((2, _C), jnp.int32),
            pltpu.VMEM((2, _C), jnp.int32),
            pltpu.VMEM((2, _C), jnp.int32),
            pltpu.VMEM((2, _C), jnp.int32),
            pltpu.VMEM((_C, d_ext), jnp.float32),
            pltpu.VMEM((_C, d_ext), jnp.float32),
            pltpu.VMEM((_C, d_ext), jnp.float32),
            pltpu.VMEM_SHARED((_NACC, d_ext), jnp.float32),
            pltpu.SemaphoreType.DMA,
            pltpu.SemaphoreType.DMA,
            pltpu.SemaphoreType.DMA,
            pltpu.SemaphoreType.DMA,
            pltpu.SemaphoreType.DMA,
            pltpu.SemaphoreType.DMA,
            pltpu.SemaphoreType.DMA,
        ],
        compiler_params=pltpu.CompilerParams(use_tc_tiling_on_sc=False),
    )(feat, edge_r)


_BLK = 512
_INV_STD = 1.0 / math.sqrt(1.0 + _EPS)


def _l1_body(p_ref, x_ref, wn_ref, wr_ref, b_ref, g_ref, be_ref,
             h_ref, rdeg_ref):
    acc = p_ref[...]
    rdeg = 1.0 / jnp.maximum(acc[:, _D:_D + 1], 1.0)
    mean = acc[:, :_D] * rdeg
    z = (jnp.dot(mean, wn_ref[...], preferred_element_type=jnp.float32)
         + jnp.dot(x_ref[...], wr_ref[...], preferred_element_type=jnp.float32)
         + b_ref[...])
    z = z * (_INV_STD * g_ref[...]) + be_ref[...]
    h_ref[...] = jnp.maximum(z, 0.0)
    rdeg_ref[...] = jnp.broadcast_to(rdeg, (_BLK, _D))


def _tc_layer1(p, x_pad, w1nT, w1rT, b1, gamma1, beta1):
    return pl.pallas_call(
        _l1_body,
        grid=(_NACC // _BLK,),
        in_specs=[
            pl.BlockSpec((_BLK, _D + 16), lambda i: (i, 0)),
            pl.BlockSpec((_BLK, _D), lambda i: (i, 0)),
            pl.BlockSpec((_D, _D), lambda i: (0, 0)),
            pl.BlockSpec((_D, _D), lambda i: (0, 0)),
            pl.BlockSpec((1, _D), lambda i: (0, 0)),
            pl.BlockSpec((1, _D), lambda i: (0, 0)),
            pl.BlockSpec((1, _D), lambda i: (0, 0)),
        ],
        out_specs=[
            pl.BlockSpec((_BLK, _D), lambda i: (i, 0)),
            pl.BlockSpec((_BLK, _D), lambda i: (i, 0)),
        ],
        out_shape=[
            jax.ShapeDtypeStruct((_NACC, _D), jnp.float32),
            jax.ShapeDtypeStruct((_NACC, _D), jnp.float32),
        ],
    )(p, x_pad, w1nT, w1rT, b1, gamma1, beta1)


def _l2_body(p_ref, h_ref, rdeg_ref, wn_ref, wr_ref, b_ref, out_ref):
    mean = p_ref[...] * rdeg_ref[...]
    z = (jnp.dot(mean, wn_ref[...], preferred_element_type=jnp.float32)
         + jnp.dot(h_ref[...], wr_ref[...], preferred_element_type=jnp.float32)
         + b_ref[...])
    m = jnp.max(z, axis=1, keepdims=True)
    s = jnp.sum(jnp.exp(z - m), axis=1, keepdims=True)
    out_ref[...] = z - m - jnp.log(s)


def _tc_layer2(p2, h, rdeg, w2nT, w2rT, b2):
    return pl.pallas_call(
        _l2_body,
        grid=(_NACC // _BLK,),
        in_specs=[
            pl.BlockSpec((_BLK, _D), lambda i: (i, 0)),
            pl.BlockSpec((_BLK, _D), lambda i: (i, 0)),
            pl.BlockSpec((_BLK, _D), lambda i: (i, 0)),
            pl.BlockSpec((_D, _D), lambda i: (0, 0)),
            pl.BlockSpec((_D, _D), lambda i: (0, 0)),
            pl.BlockSpec((1, _D), lambda i: (0, 0)),
        ],
        out_specs=pl.BlockSpec((_BLK, _D), lambda i: (i, 0)),
        out_shape=jax.ShapeDtypeStruct((_NACC, _D), jnp.float32),
    )(p2, h, rdeg, w2nT, w2rT, b2)


def kernel(x, edge_index, W1n, W1r, b1, gamma1, beta1, W2n, W2r, b2):
    src = edge_index[0]
    dst = edge_index[1]
    pad = _EPAD - _E
    src_r = jnp.concatenate(
        [src, jnp.zeros((pad,), jnp.int32)]).reshape(_NCHUNKS, _C)
    dst_r = jnp.concatenate(
        [dst, jnp.full((pad,), _N, jnp.int32)]).reshape(_NCHUNKS, _C)
    edge_r = jnp.stack([src_r, dst_r], axis=1)
    x_ext = jnp.concatenate(
        [x, jnp.ones((_N, 1), jnp.float32), jnp.zeros((_N, 15), jnp.float32)],
        axis=1)

    p1 = _sc_segment_sum(x_ext, edge_r, _D + 16)
    x_pad = jnp.pad(x, ((0, _NACC - _N), (0, 0)))
    h, rdeg = _tc_layer1(p1, x_pad, W1n.T, W1r.T, b1.reshape(1, _D),
                         gamma1.reshape(1, _D), beta1.reshape(1, _D))
    p2 = _sc_segment_sum(h, edge_r, _D)
    out = _tc_layer2(p2, h, rdeg, W2n.T, W2r.T, b2.reshape(1, _D))
    return out[:_N]
